# Initial kernel scaffold; baseline (speedup 1.0000x reference)
import functools

import jax
import jax.numpy as jnp
from jax import lax
from jax.experimental import pallas as pl
from jax.experimental.pallas import tpu as pltpu
from jax.experimental.pallas import tpu_sc as plsc

N_NODES = 10000
N_EDGES = 320000
D = 128

CHUNK = 128
NW = 32
N_CHUNKS = N_EDGES // CHUNK
CHUNKS_PER_W = -(-N_CHUNKS // NW)
DEGW = 16
ZR = 25

_mesh = plsc.VectorSubcoreMesh(core_axis_name="c", subcore_axis_name="s")


def _sc_agg_body(compute_deg, x_hbm, src_hbm, dst_hbm, *refs):
    if compute_deg:
        agg_out, deg_out, src_v, dst_v, rows_v, zbuf, acc_sh, ones_v, zdeg, deg_sh = refs
    else:
        agg_out, src_v, dst_v, rows_v, zbuf, acc_sh = refs
    c = lax.axis_index("c")
    s = lax.axis_index("s")
    wid = s * 2 + c

    for r in range(ZR):
        for cc in range(D // 16):
            zbuf[r, pl.ds(cc * 16, 16)] = jnp.zeros((16,), jnp.float32)
    if compute_deg:
        for r in range(ZR):
            zdeg[r, pl.ds(0, 16)] = jnp.zeros((16,), jnp.float32)
        for r in range(CHUNK):
            ones_v[r, pl.ds(0, 16)] = jnp.ones((16,), jnp.float32)

    @pl.loop(0, N_NODES // ZR // 16)
    def _(i):
        row0 = (s * (N_NODES // ZR // 16) + i) * ZR
        pltpu.sync_copy(zbuf, acc_sh.at[pl.ds(row0, ZR)])
        if compute_deg:
            pltpu.sync_copy(zdeg, deg_sh.at[pl.ds(row0, ZR)])

    plsc.subcore_barrier()

    @pl.loop(0, CHUNKS_PER_W)
    def _(i):
        g = wid + i * NW

        @pl.when(g < N_CHUNKS)
        def _():
            base = g * CHUNK
            pltpu.sync_copy(src_hbm.at[pl.ds(base, CHUNK)], src_v.at[0])
            pltpu.sync_copy(dst_hbm.at[pl.ds(base, CHUNK)], dst_v.at[0])
            pltpu.sync_copy(x_hbm.at[src_v.at[0]], rows_v)
            pltpu.sync_copy(rows_v, acc_sh.at[dst_v.at[0]], add=True)
            if compute_deg:
                pltpu.sync_copy(ones_v, deg_sh.at[dst_v.at[0]], add=True)

    plsc.subcore_barrier()

    rows_per_sub = N_NODES // 16

    @pl.loop(0, rows_per_sub // ZR)
    def _(i):
        row0 = s * rows_per_sub + i * ZR
        pltpu.sync_copy(acc_sh.at[pl.ds(row0, ZR)], agg_out.at[c].at[pl.ds(row0, ZR)])
        if compute_deg:
            pltpu.sync_copy(deg_sh.at[pl.ds(row0, ZR)], deg_out.at[c].at[pl.ds(row0, ZR)])


def _make_sc_agg(compute_deg):
    out_type = [jax.ShapeDtypeStruct((2, N_NODES, D), jnp.float32)]
    scratch = [
        pltpu.VMEM((1, CHUNK), jnp.int32),
        pltpu.VMEM((1, CHUNK), jnp.int32),
        pltpu.VMEM((CHUNK, D), jnp.float32),
        pltpu.VMEM((ZR, D), jnp.float32),
        pltpu.VMEM_SHARED((N_NODES, D), jnp.float32),
    ]
    if compute_deg:
        out_type.append(jax.ShapeDtypeStruct((2, N_NODES, DEGW), jnp.float32))
        scratch += [
            pltpu.VMEM((CHUNK, DEGW), jnp.float32),
            pltpu.VMEM((ZR, DEGW), jnp.float32),
            pltpu.VMEM_SHARED((N_NODES, DEGW), jnp.float32),
        ]
    return pl.kernel(
        functools.partial(_sc_agg_body, compute_deg),
        out_type=out_type,
        mesh=_mesh,
        scratch_types=scratch,
    )


_sc_agg_deg = _make_sc_agg(True)
_sc_agg = _make_sc_agg(False)


_BLK = 1000


def _linear_body(x_ref, w_ref, b_ref, o_ref):
    o_ref[...] = (
        jnp.dot(x_ref[...], w_ref[...], preferred_element_type=jnp.float32)
        + b_ref[...]
    )


def _tc_linear(x, w, b):
    return pl.pallas_call(
        _linear_body,
        grid=(N_NODES // _BLK,),
        in_specs=[
            pl.BlockSpec((_BLK, D), lambda i: (i, 0)),
            pl.BlockSpec((D, D), lambda i: (0, 0)),
            pl.BlockSpec((1, D), lambda i: (0, 0)),
        ],
        out_specs=pl.BlockSpec((_BLK, D), lambda i: (i, 0)),
        out_shape=jax.ShapeDtypeStruct((N_NODES, D), jnp.float32),
    )(x, w, b.reshape(1, D))


def _combine_body(final, agg_ref, deg_ref, xr_ref, wl_ref, wlin_ref, blin_ref, o_ref):
    a = agg_ref[0] + agg_ref[1]
    d = jnp.maximum(deg_ref[0, :, 0] + deg_ref[1, :, 0], 1.0)
    mean = a / d[:, None]
    h = jnp.maximum(
        jnp.dot(mean, wl_ref[...], preferred_element_type=jnp.float32) + xr_ref[...],
        0.0,
    )
    if final:
        o_ref[...] = (
            jnp.dot(h, wlin_ref[...], preferred_element_type=jnp.float32)
            + blin_ref[...]
        )
    else:
        o_ref[...] = h


def _tc_combine(agg, deg, xr, wl):
    def body(agg_ref, deg_ref, xr_ref, wl_ref, o_ref):
        _combine_body(False, agg_ref, deg_ref, xr_ref, wl_ref, None, None, o_ref)

    return pl.pallas_call(
        body,
        grid=(N_NODES // _BLK,),
        in_specs=[
            pl.BlockSpec((2, _BLK, D), lambda i: (0, i, 0)),
            pl.BlockSpec((2, _BLK, DEGW), lambda i: (0, i, 0)),
            pl.BlockSpec((_BLK, D), lambda i: (i, 0)),
            pl.BlockSpec((D, D), lambda i: (0, 0)),
        ],
        out_specs=pl.BlockSpec((_BLK, D), lambda i: (i, 0)),
        out_shape=jax.ShapeDtypeStruct((N_NODES, D), jnp.float32),
    )(agg, deg, xr, wl)


def _tc_combine_final(agg, deg, xr, wl, wlin, blin):
    return pl.pallas_call(
        functools.partial(_combine_body, True),
        grid=(N_NODES // _BLK,),
        in_specs=[
            pl.BlockSpec((2, _BLK, D), lambda i: (0, i, 0)),
            pl.BlockSpec((2, _BLK, DEGW), lambda i: (0, i, 0)),
            pl.BlockSpec((_BLK, D), lambda i: (i, 0)),
            pl.BlockSpec((D, D), lambda i: (0, 0)),
            pl.BlockSpec((D, D), lambda i: (0, 0)),
            pl.BlockSpec((1, D), lambda i: (0, 0)),
        ],
        out_specs=pl.BlockSpec((_BLK, D), lambda i: (i, 0)),
        out_shape=jax.ShapeDtypeStruct((N_NODES, D), jnp.float32),
    )(agg, deg, xr, wl, wlin, blin.reshape(1, D))


def kernel(x, edge_index, Wl1, bl1, Wr1, Wl2, bl2, Wr2, Wlin, blin):
    src = edge_index[0].astype(jnp.int32)
    dst = edge_index[1].astype(jnp.int32)

    agg1, deg = _sc_agg_deg(x, src, dst)
    xr1 = _tc_linear(x, Wr1, bl1)
    h1 = _tc_combine(agg1, deg, xr1, Wl1)

    (agg2,) = _sc_agg(h1, src, dst)
    xr2 = _tc_linear(h1, Wr2, bl2)
    return _tc_combine_final(agg2, deg, xr2, Wl2, Wlin, blin)


# speedup vs baseline: 5.8369x; 5.8369x over previous
import functools

import jax
import jax.numpy as jnp
from jax import lax
from jax.experimental import pallas as pl
from jax.experimental.pallas import tpu as pltpu
from jax.experimental.pallas import tpu_sc as plsc

N_NODES = 10000
N_EDGES = 320000
D = 128

CHUNK = 128
NW = 32
N_CHUNKS = N_EDGES // CHUNK
CHUNKS_PER_W = -(-N_CHUNKS // NW)
DEGW = 128
ZR = 40
NZB = N_NODES // ZR

@functools.lru_cache(maxsize=None)
def _sc_mesh():
    return plsc.VectorSubcoreMesh(core_axis_name="c", subcore_axis_name="s")


def _fill_const(ref, nrows, val):
    @pl.loop(0, nrows)
    def _(r):
        for cc in range(D // 16):
            ref[r, pl.ds(cc * 16, 16)] = jnp.full((16,), val, jnp.float32)


def _sc_agg_body(x_hbm, src_hbm, dst_hbm, agg_out, src_v, dst_v, rows_v, zbuf, acc_sh):
    c = lax.axis_index("c")
    s = lax.axis_index("s")
    wid = s * 2 + c

    _fill_const(zbuf, ZR, 0.0)

    @pl.loop(0, -(-NZB // 16))
    def _(i):
        b = s + 16 * i

        @pl.when(b < NZB)
        def _():
            pltpu.sync_copy(zbuf, acc_sh.at[pl.ds(b * ZR, ZR)])

    plsc.subcore_barrier()

    @pl.loop(0, CHUNKS_PER_W)
    def _(i):
        g = wid + i * NW

        @pl.when(g < N_CHUNKS)
        def _():
            base = g * CHUNK
            pltpu.sync_copy(src_hbm.at[pl.ds(base, CHUNK)], src_v.at[0])
            pltpu.sync_copy(dst_hbm.at[pl.ds(base, CHUNK)], dst_v.at[0])
            pltpu.sync_copy(x_hbm.at[src_v.at[0]], rows_v)
            pltpu.sync_copy(rows_v, acc_sh.at[dst_v.at[0]], add=True)

    plsc.subcore_barrier()

    @pl.loop(0, -(-NZB // 16))
    def _(i):
        b = s + 16 * i

        @pl.when(b < NZB)
        def _():
            pltpu.sync_copy(
                acc_sh.at[pl.ds(b * ZR, ZR)], agg_out.at[c].at[pl.ds(b * ZR, ZR)]
            )


def _sc_deg_body(dst_hbm, deg_out, dst_v, ones_v, zdeg, deg_sh):
    c = lax.axis_index("c")
    s = lax.axis_index("s")
    wid = s * 2 + c

    _fill_const(zdeg, ZR, 0.0)
    _fill_const(ones_v, CHUNK, 1.0)

    @pl.loop(0, -(-NZB // 16))
    def _(i):
        b = s + 16 * i

        @pl.when(b < NZB)
        def _():
            pltpu.sync_copy(zdeg, deg_sh.at[pl.ds(b * ZR, ZR)])

    plsc.subcore_barrier()

    @pl.loop(0, CHUNKS_PER_W)
    def _(i):
        g = wid + i * NW

        @pl.when(g < N_CHUNKS)
        def _():
            pltpu.sync_copy(dst_hbm.at[pl.ds(g * CHUNK, CHUNK)], dst_v.at[0])
            pltpu.sync_copy(ones_v, deg_sh.at[dst_v.at[0]], add=True)

    plsc.subcore_barrier()

    @pl.loop(0, -(-NZB // 16))
    def _(i):
        b = s + 16 * i

        @pl.when(b < NZB)
        def _():
            pltpu.sync_copy(
                deg_sh.at[pl.ds(b * ZR, ZR)], deg_out.at[c].at[pl.ds(b * ZR, ZR)]
            )


@functools.lru_cache(maxsize=None)
def _sc_agg():
    return pl.kernel(
        _sc_agg_body,
        out_type=jax.ShapeDtypeStruct((2, N_NODES, D), jnp.float32),
        mesh=_sc_mesh(),
        scratch_types=[
            pltpu.VMEM((1, CHUNK), jnp.int32),
            pltpu.VMEM((1, CHUNK), jnp.int32),
            pltpu.VMEM((CHUNK, D), jnp.float32),
            pltpu.VMEM((ZR, D), jnp.float32),
            pltpu.VMEM_SHARED((N_NODES, D), jnp.float32),
        ],
    )


@functools.lru_cache(maxsize=None)
def _sc_deg():
    return pl.kernel(
        _sc_deg_body,
        out_type=jax.ShapeDtypeStruct((2, N_NODES, DEGW), jnp.float32),
        mesh=_sc_mesh(),
        scratch_types=[
            pltpu.VMEM((1, CHUNK), jnp.int32),
            pltpu.VMEM((CHUNK, DEGW), jnp.float32),
            pltpu.VMEM((ZR, DEGW), jnp.float32),
            pltpu.VMEM_SHARED((N_NODES, DEGW), jnp.float32),
        ],
    )


_BLK = 1000


def _linear_body(x_ref, w_ref, b_ref, o_ref):
    o_ref[...] = (
        jnp.dot(x_ref[...], w_ref[...], preferred_element_type=jnp.float32)
        + b_ref[...]
    )


def _tc_linear(x, w, b):
    return pl.pallas_call(
        _linear_body,
        grid=(N_NODES // _BLK,),
        in_specs=[
            pl.BlockSpec((_BLK, D), lambda i: (i, 0)),
            pl.BlockSpec((D, D), lambda i: (0, 0)),
            pl.BlockSpec((1, D), lambda i: (0, 0)),
        ],
        out_specs=pl.BlockSpec((_BLK, D), lambda i: (i, 0)),
        out_shape=jax.ShapeDtypeStruct((N_NODES, D), jnp.float32),
    )(x, w, b.reshape(1, D))


def _combine_body(final, agg_ref, deg_ref, xr_ref, wl_ref, wlin_ref, blin_ref, o_ref):
    a = agg_ref[0] + agg_ref[1]
    d = jnp.maximum(deg_ref[0, :, 0] + deg_ref[1, :, 0], 1.0)
    mean = a / d[:, None]
    h = jnp.maximum(
        jnp.dot(mean, wl_ref[...], preferred_element_type=jnp.float32) + xr_ref[...],
        0.0,
    )
    if final:
        o_ref[...] = (
            jnp.dot(h, wlin_ref[...], preferred_element_type=jnp.float32)
            + blin_ref[...]
        )
    else:
        o_ref[...] = h


def _tc_combine(agg, deg, xr, wl):
    def body(agg_ref, deg_ref, xr_ref, wl_ref, o_ref):
        _combine_body(False, agg_ref, deg_ref, xr_ref, wl_ref, None, None, o_ref)

    return pl.pallas_call(
        body,
        grid=(N_NODES // _BLK,),
        in_specs=[
            pl.BlockSpec((2, _BLK, D), lambda i: (0, i, 0)),
            pl.BlockSpec((2, _BLK, DEGW), lambda i: (0, i, 0)),
            pl.BlockSpec((_BLK, D), lambda i: (i, 0)),
            pl.BlockSpec((D, D), lambda i: (0, 0)),
        ],
        out_specs=pl.BlockSpec((_BLK, D), lambda i: (i, 0)),
        out_shape=jax.ShapeDtypeStruct((N_NODES, D), jnp.float32),
    )(agg, deg, xr, wl)


def _tc_combine_final(agg, deg, xr, wl, wlin, blin):
    return pl.pallas_call(
        functools.partial(_combine_body, True),
        grid=(N_NODES // _BLK,),
        in_specs=[
            pl.BlockSpec((2, _BLK, D), lambda i: (0, i, 0)),
            pl.BlockSpec((2, _BLK, DEGW), lambda i: (0, i, 0)),
            pl.BlockSpec((_BLK, D), lambda i: (i, 0)),
            pl.BlockSpec((D, D), lambda i: (0, 0)),
            pl.BlockSpec((D, D), lambda i: (0, 0)),
            pl.BlockSpec((1, D), lambda i: (0, 0)),
        ],
        out_specs=pl.BlockSpec((_BLK, D), lambda i: (i, 0)),
        out_shape=jax.ShapeDtypeStruct((N_NODES, D), jnp.float32),
    )(agg, deg, xr, wl, wlin, blin.reshape(1, D))


def kernel(x, edge_index, Wl1, bl1, Wr1, Wl2, bl2, Wr2, Wlin, blin):
    src = edge_index[0].astype(jnp.int32)
    dst = edge_index[1].astype(jnp.int32)

    deg = _sc_deg()(dst)
    agg1 = _sc_agg()(x, src, dst)
    xr1 = _tc_linear(x, Wr1, bl1)
    h1 = _tc_combine(agg1, deg, xr1, Wl1)

    agg2 = _sc_agg()(h1, src, dst)
    xr2 = _tc_linear(h1, Wr2, bl2)
    return _tc_combine_final(agg2, deg, xr2, Wl2, Wlin, blin)
